# baseline (speedup 1.0000x reference)
import functools

import jax
import jax.numpy as jnp
from jax import lax
from jax.experimental import pallas as pl
from jax.experimental.pallas import tpu as pltpu
from jax.experimental.pallas import tpu_sc as plsc

_MAX_LEN = 8192
_D_MODEL = 768
_SC_ROWS = _MAX_LEN // 2
_CHUNK_ROWS = 512
_NUM_CHUNKS = _SC_ROWS // _CHUNK_ROWS
_NBUF = 4
_NQ = 2


@functools.partial(
    pl.kernel,
    out_type=jax.ShapeDtypeStruct((_MAX_LEN, _D_MODEL), jnp.float32),
    mesh=plsc.VectorSubcoreMesh(core_axis_name="c", subcore_axis_name="s"),
    scratch_types=[
        pltpu.VMEM_SHARED((_NBUF, _CHUNK_ROWS, _D_MODEL), jnp.float32),
    ]
    + [pltpu.SemaphoreType.DMA] * (2 * _NQ),
)
def _pos_encoding_copy(emb_hbm, out_hbm, spbuf, *sems):
    in_sems, out_sems = sems[:_NQ], sems[_NQ:]
    base = lax.axis_index("c") * _SC_ROWS

    def copy_in(j):
        return pltpu.async_copy(
            emb_hbm.at[pl.ds(base + j * _CHUNK_ROWS, _CHUNK_ROWS)],
            spbuf.at[j % _NBUF],
            in_sems[j % _NQ],
        )

    def copy_out(j):
        return pltpu.async_copy(
            spbuf.at[j % _NBUF],
            out_hbm.at[pl.ds(base + j * _CHUNK_ROWS, _CHUNK_ROWS)],
            out_sems[j % _NQ],
        )

    @pl.when(lax.axis_index("s") == 0)
    def _():
        ins = [None] * _NUM_CHUNKS
        outs = [None] * _NUM_CHUNKS
        for j in range(_NBUF):
            ins[j] = copy_in(j)
        for j in range(_NUM_CHUNKS):
            ins[j].wait()
            outs[j] = copy_out(j)
            nxt = j + _NBUF
            if nxt < _NUM_CHUNKS:
                outs[j].wait()
                ins[nxt] = copy_in(nxt)
        for j in range(_NUM_CHUNKS - _NBUF, _NUM_CHUNKS):
            if j >= 0:
                outs[j].wait()


def kernel(x, embedding):
    del x
    return _pos_encoding_copy(embedding)[None]


# speedup vs baseline: 1.1292x; 1.1292x over previous
import functools

import jax
import jax.numpy as jnp
from jax import lax
from jax.experimental import pallas as pl
from jax.experimental.pallas import tpu as pltpu
from jax.experimental.pallas import tpu_sc as plsc

_MAX_LEN = 8192
_D_MODEL = 768
_NUM_WORKERS = 32
_ROWS_PER_WORKER = _MAX_LEN // _NUM_WORKERS


_CHUNK_ROWS = 32
_NUM_CHUNKS = _ROWS_PER_WORKER // _CHUNK_ROWS
_NBUF = 4
_NQ = 2


@functools.partial(
    pl.kernel,
    out_type=jax.ShapeDtypeStruct((_MAX_LEN, _D_MODEL), jnp.float32),
    mesh=plsc.VectorSubcoreMesh(core_axis_name="c", subcore_axis_name="s"),
)
def _pos_encoding_copy(emb_hbm, out_hbm):
    pl.run_scoped(
        functools.partial(_worker_body, emb_hbm, out_hbm),
        pltpu.VMEM((_NBUF, _CHUNK_ROWS, _D_MODEL), jnp.float32),
        *([pltpu.SemaphoreType.DMA] * (2 * _NQ)),
    )


def _worker_body(emb_hbm, out_hbm, buf, *sems):
    in_sems, out_sems = sems[:_NQ], sems[_NQ:]
    wid = lax.axis_index("c") * _NUM_WORKERS // 2 + lax.axis_index("s")
    base = wid * _ROWS_PER_WORKER

    def copy_in(j):
        return pltpu.async_copy(
            emb_hbm.at[pl.ds(base + j * _CHUNK_ROWS, _CHUNK_ROWS)],
            buf.at[j % _NBUF],
            in_sems[j % _NQ],
        )

    def copy_out(j):
        return pltpu.async_copy(
            buf.at[j % _NBUF],
            out_hbm.at[pl.ds(base + j * _CHUNK_ROWS, _CHUNK_ROWS)],
            out_sems[j % _NQ],
        )

    ins = [None] * _NUM_CHUNKS
    outs = [None] * _NUM_CHUNKS
    for j in range(_NBUF):
        ins[j] = copy_in(j)
    for j in range(_NUM_CHUNKS):
        ins[j].wait()
        outs[j] = copy_out(j)
        nxt = j + _NBUF
        if nxt < _NUM_CHUNKS:
            outs[j].wait()
            ins[nxt] = copy_in(nxt)
    for j in range(_NUM_CHUNKS - _NBUF, _NUM_CHUNKS):
        if j >= 0:
            outs[j].wait()


def kernel(x, embedding):
    del x
    return _pos_encoding_copy(embedding)[None]
